# Initial kernel scaffold; baseline (speedup 1.0000x reference)
import jax
import jax.numpy as jnp
from jax.experimental import pallas as pl


def kernel(x_nodes, block0_edge_index, block1_edge_index, pair_u, pair_v, e_feat, W_self0, W_neigh0, b0, gamma0, beta0, W_self1, W_neigh1, b1, gamma1, beta1, W_mlp1, b_mlp1, W_mlp2, b_mlp2):
    raise NotImplementedError("write your pallas kernel here")


# speedup vs baseline: 3.7875x; 3.7875x over previous
import functools

import jax
import jax.numpy as jnp
from jax import lax
from jax.experimental import pallas as pl
from jax.experimental.pallas import tpu as pltpu
from jax.experimental.pallas import tpu_sc as plsc

N_NODES = 10000
N_EDGES = 320000
N_PAIRS = 100000
D = 128
EDGE_IN = 16
N_CLS = 8
EPS = 1e-5

NC = 2
NS = 16
NW = NC * NS

EPT = N_EDGES // NW
CH = 80
CPT = EPT // CH
ROWS_PAD = 10240
RPT = ROWS_PAD // NS

DEPT = N_EDGES // NS
DCPT = DEPT // CH

P_PAD = 102400
PPT = P_PAD // NW
GCH = 128
NGCH = PPT // GCH

_sc_mesh = plsc.VectorSubcoreMesh(core_axis_name="c", subcore_axis_name="s")


def _fill_wide(ref, n_rows, val):
    def body(k, carry):
        ref[k // 8, pl.ds((k % 8) * 16, 16)] = jnp.full((16,), val,
                                                        jnp.float32)
        return carry
    lax.fori_loop(0, n_rows * 8, body, 0)


def _make_sc_degrees():
    @functools.partial(
        pl.kernel,
        out_type=[
            jax.ShapeDtypeStruct((ROWS_PAD,), jnp.float32),
            jax.ShapeDtypeStruct((ROWS_PAD,), jnp.float32),
        ],
        mesh=_sc_mesh,
        scratch_types=[
            pltpu.VMEM((DEPT,), jnp.int32),
            pltpu.VMEM((ROWS_PAD,), jnp.float32),
            pltpu.VMEM((RPT,), jnp.float32),
            pltpu.VMEM((RPT,), jnp.float32),
            pltpu.VMEM_SHARED((NS, ROWS_PAD), jnp.float32),
        ],
        compiler_params=pltpu.CompilerParams(needs_layout_passes=False),
    )
    def sc_degrees(dst0_hbm, dst1_hbm, deg0_out, deg1_out,
                   dstv, histv, acc, tmp, stage_sh):
        cid = lax.axis_index("c")
        sid = lax.axis_index("s")

        def zbody(i, carry):
            histv[pl.ds(i * 16, 16)] = jnp.zeros((16,), jnp.float32)
            return carry
        lax.fori_loop(0, ROWS_PAD // 16, zbody, 0)

        @pl.when(cid == 0)
        def _():
            pltpu.sync_copy(dst0_hbm.at[sid], dstv)

        @pl.when(cid == 1)
        def _():
            pltpu.sync_copy(dst1_hbm.at[sid], dstv)

        ones = jnp.ones((16,), jnp.float32)

        def hbody(k, carry):
            idx = dstv[pl.ds(k * 16, 16)]
            plsc.addupdate_scatter(histv, [idx], ones)
            return carry
        lax.fori_loop(0, DEPT // 16, hbody, 0)

        pltpu.sync_copy(histv, stage_sh.at[sid])
        plsc.subcore_barrier()

        pltpu.sync_copy(stage_sh.at[0, pl.ds(sid * RPT, RPT)], acc)
        for j in range(1, NS):
            pltpu.sync_copy(stage_sh.at[j, pl.ds(sid * RPT, RPT)], tmp)

            def abody(i, carry):
                acc[pl.ds(i * 16, 16)] = (acc[pl.ds(i * 16, 16)]
                                          + tmp[pl.ds(i * 16, 16)])
                return carry
            lax.fori_loop(0, RPT // 16, abody, 0)

        @pl.when(cid == 0)
        def _():
            pltpu.sync_copy(acc, deg0_out.at[pl.ds(sid * RPT, RPT)])

        @pl.when(cid == 1)
        def _():
            pltpu.sync_copy(acc, deg1_out.at[pl.ds(sid * RPT, RPT)])

    return sc_degrees


_sc_degrees = _make_sc_degrees()


def _make_sc_aggregate():
    @functools.partial(
        pl.kernel,
        out_type=jax.ShapeDtypeStruct((NC, ROWS_PAD, D), jnp.float32),
        mesh=_sc_mesh,
        scratch_types=[
            pltpu.VMEM((EPT,), jnp.int32),
            pltpu.VMEM((CH,), jnp.int32),
            pltpu.VMEM((CH, D), jnp.float32),
            pltpu.VMEM((CH, D), jnp.float32),
            pltpu.VMEM_SHARED((ROWS_PAD, D), jnp.float32),
            pltpu.SemaphoreType.DMA,
        ],
    )
    def sc_aggregate(h_hbm, src_hbm, dst_hbm, agg_out,
                     srcv, dstv, rows_v, zw_v, agg_sh, sem):
        cid = lax.axis_index("c")
        sid = lax.axis_index("s")
        wid = cid * NS + sid

        _fill_wide(zw_v, CH, 0.0)
        pltpu.sync_copy(src_hbm.at[wid], srcv)
        for j in range(RPT // CH):
            pltpu.sync_copy(zw_v, agg_sh.at[pl.ds(sid * RPT + j * CH, CH)])
        plsc.subcore_barrier()

        def chunk(c, carry):
            pltpu.sync_copy(dst_hbm.at[wid * CPT + c], dstv)
            pltpu.async_copy(h_hbm.at[srcv.at[pl.ds(c * CH, CH)]],
                             rows_v, sem).wait()
            pltpu.sync_copy(rows_v, agg_sh.at[dstv], add=True)
            return carry

        lax.fori_loop(0, CPT, chunk, 0)
        plsc.subcore_barrier()

        pltpu.sync_copy(agg_sh.at[pl.ds(sid * RPT, RPT)],
                        agg_out.at[cid, pl.ds(sid * RPT, RPT)])

    return sc_aggregate


_sc_aggregate = _make_sc_aggregate()


def _make_sc_pair_gather():
    @functools.partial(
        pl.kernel,
        out_type=[
            jax.ShapeDtypeStruct((P_PAD, D), jnp.float32),
            jax.ShapeDtypeStruct((P_PAD, D), jnp.float32),
        ],
        mesh=_sc_mesh,
        scratch_types=[
            pltpu.VMEM((PPT,), jnp.int32),
            pltpu.VMEM((PPT,), jnp.int32),
            pltpu.VMEM((GCH, D), jnp.float32),
            pltpu.SemaphoreType.DMA,
        ],
    )
    def sc_pair_gather(h_hbm, pu_hbm, pv_hbm, hu_out, hv_out,
                       puv, pvv, rows_v, sem):
        cid = lax.axis_index("c")
        sid = lax.axis_index("s")
        wid = cid * NS + sid

        pltpu.sync_copy(pu_hbm.at[wid], puv)
        pltpu.sync_copy(pv_hbm.at[wid], pvv)

        def chunk(c, carry):
            base = wid * PPT + c * GCH
            pltpu.async_copy(h_hbm.at[puv.at[pl.ds(c * GCH, GCH)]],
                             rows_v, sem).wait()
            pltpu.sync_copy(rows_v, hu_out.at[pl.ds(base, GCH)])
            pltpu.async_copy(h_hbm.at[pvv.at[pl.ds(c * GCH, GCH)]],
                             rows_v, sem).wait()
            pltpu.sync_copy(rows_v, hv_out.at[pl.ds(base, GCH)])
            return carry

        lax.fori_loop(0, NGCH, chunk, 0)

    return sc_pair_gather


_sc_pair_gather = _make_sc_pair_gather()


def _layer_body(h_ref, aggp_ref, deg_ref, ws_ref, wn_ref, b_ref,
                g_ref, be_ref, out_ref):
    deg = jnp.maximum(deg_ref[...], 1.0)
    agg = aggp_ref[0, 0:N_NODES, :] + aggp_ref[1, 0:N_NODES, :]
    mean = agg / deg
    z = (jnp.dot(h_ref[...], ws_ref[...], preferred_element_type=jnp.float32)
         + jnp.dot(mean, wn_ref[...], preferred_element_type=jnp.float32)
         + b_ref[...])
    mu = jnp.mean(z, axis=0, keepdims=True)
    var = jnp.mean((z - mu) * (z - mu), axis=0, keepdims=True)
    bn = g_ref[...] * (z - mu) / jnp.sqrt(var + EPS) + be_ref[...]
    out_ref[...] = jnp.maximum(bn, 0.0)


def _tc_layer(h, aggp, deg, w_self, w_neigh, b, gamma, beta):
    return pl.pallas_call(
        _layer_body,
        out_shape=jax.ShapeDtypeStruct((N_NODES, D), jnp.float32),
    )(h, aggp, deg, w_self, w_neigh, b, gamma, beta)


MLP_BLK = 2048


def _mlp_body(hu_ref, hv_ref, ef_ref, wa_ref, wb_ref, wc_ref, b1_ref,
              w2_ref, b2_ref, out_ref):
    z = (jnp.dot(hu_ref[...], wa_ref[...], preferred_element_type=jnp.float32)
         + jnp.dot(hv_ref[...], wb_ref[...], preferred_element_type=jnp.float32)
         + jnp.dot(ef_ref[...], wc_ref[...], preferred_element_type=jnp.float32)
         + b1_ref[...])
    z = jnp.maximum(z, 0.0)
    out_ref[...] = (jnp.dot(z, w2_ref[...], preferred_element_type=jnp.float32)
                    + b2_ref[...])


def _tc_mlp(hu, hv, ef, wa, wb, wc, b1, w2, b2):
    n_blk = P_PAD // MLP_BLK
    return pl.pallas_call(
        _mlp_body,
        grid=(n_blk,),
        in_specs=[
            pl.BlockSpec((MLP_BLK, D), lambda i: (i, 0)),
            pl.BlockSpec((MLP_BLK, D), lambda i: (i, 0)),
            pl.BlockSpec((MLP_BLK, EDGE_IN), lambda i: (i, 0)),
            pl.BlockSpec((D, D), lambda i: (0, 0)),
            pl.BlockSpec((D, D), lambda i: (0, 0)),
            pl.BlockSpec((EDGE_IN, D), lambda i: (0, 0)),
            pl.BlockSpec((1, D), lambda i: (0, 0)),
            pl.BlockSpec((D, N_CLS), lambda i: (0, 0)),
            pl.BlockSpec((1, N_CLS), lambda i: (0, 0)),
        ],
        out_specs=pl.BlockSpec((MLP_BLK, N_CLS), lambda i: (i, 0)),
        out_shape=jax.ShapeDtypeStruct((P_PAD, N_CLS), jnp.float32),
    )(hu, hv, ef, wa, wb, wc, b1, w2, b2)


def kernel(x_nodes, block0_edge_index, block1_edge_index, pair_u, pair_v,
           e_feat, W_self0, W_neigh0, b0, gamma0, beta0,
           W_self1, W_neigh1, b1, gamma1, beta1,
           W_mlp1, b_mlp1, W_mlp2, b_mlp2):
    src0 = block0_edge_index[0].reshape(NW, EPT)
    dst0 = block0_edge_index[1].reshape(NW * CPT, CH)
    src1 = block1_edge_index[0].reshape(NW, EPT)
    dst1 = block1_edge_index[1].reshape(NW * CPT, CH)
    dst0_d = block0_edge_index[1].reshape(NS, DEPT)
    dst1_d = block1_edge_index[1].reshape(NS, DEPT)
    pad = P_PAD - N_PAIRS
    pu = jnp.concatenate([pair_u, jnp.zeros((pad,), jnp.int32)])
    pv = jnp.concatenate([pair_v, jnp.zeros((pad,), jnp.int32)])
    pu2 = pu.reshape(NW, PPT)
    pv2 = pv.reshape(NW, PPT)
    ef = jnp.concatenate([e_feat, jnp.zeros((pad, EDGE_IN), jnp.float32)])

    deg0f, deg1f = _sc_degrees(dst0_d, dst1_d)
    deg0 = deg0f[:N_NODES, None]
    deg1 = deg1f[:N_NODES, None]

    aggp0 = _sc_aggregate(x_nodes, src0, dst0)
    h1 = _tc_layer(x_nodes, aggp0, deg0, W_self0, W_neigh0,
                   b0.reshape(1, D), gamma0.reshape(1, D), beta0.reshape(1, D))

    aggp1 = _sc_aggregate(h1, src1, dst1)
    h2 = _tc_layer(h1, aggp1, deg1, W_self1, W_neigh1,
                   b1.reshape(1, D), gamma1.reshape(1, D), beta1.reshape(1, D))

    hu, hv = _sc_pair_gather(h2, pu2, pv2)
    wa = W_mlp1[0:D]
    wb = W_mlp1[D:2 * D]
    wc = W_mlp1[2 * D:]
    out = _tc_mlp(hu, hv, ef, wa, wb, wc, b_mlp1.reshape(1, -1),
                  W_mlp2, b_mlp2.reshape(1, -1))
    return out[:N_PAIRS]


# speedup vs baseline: 5.4445x; 1.4375x over previous
import functools

import jax
import jax.numpy as jnp
from jax import lax
from jax.experimental import pallas as pl
from jax.experimental.pallas import tpu as pltpu
from jax.experimental.pallas import tpu_sc as plsc

N_NODES = 10000
N_EDGES = 320000
N_PAIRS = 100000
D = 128
EDGE_IN = 16
N_CLS = 8
EPS = 1e-5

NC = 2
NS = 16
NW = NC * NS

EPT = N_EDGES // NW
CH = 80
CPT = EPT // CH
ROWS_PAD = 10240
RPT = ROWS_PAD // NS

DEPT = N_EDGES // NS
DCPT = DEPT // CH

P_PAD = 102400
PPT = P_PAD // NW
GCH = 128
NGCH = PPT // GCH

_sc_mesh = plsc.VectorSubcoreMesh(core_axis_name="c", subcore_axis_name="s")


def _fill_wide(ref, n_rows, val):
    def body(k, carry):
        ref[k // 8, pl.ds((k % 8) * 16, 16)] = jnp.full((16,), val,
                                                        jnp.float32)
        return carry
    lax.fori_loop(0, n_rows * 8, body, 0)


def _make_sc_degrees():
    @functools.partial(
        pl.kernel,
        out_type=[
            jax.ShapeDtypeStruct((ROWS_PAD,), jnp.float32),
            jax.ShapeDtypeStruct((ROWS_PAD,), jnp.float32),
        ],
        mesh=_sc_mesh,
        scratch_types=[
            pltpu.VMEM((DEPT,), jnp.int32),
            pltpu.VMEM((ROWS_PAD,), jnp.float32),
            pltpu.VMEM((RPT,), jnp.float32),
            pltpu.VMEM((RPT,), jnp.float32),
            pltpu.VMEM_SHARED((NS, ROWS_PAD), jnp.float32),
        ],
        compiler_params=pltpu.CompilerParams(needs_layout_passes=False),
    )
    def sc_degrees(dst0_hbm, dst1_hbm, deg0_out, deg1_out,
                   dstv, histv, acc, tmp, stage_sh):
        cid = lax.axis_index("c")
        sid = lax.axis_index("s")

        def zbody(i, carry):
            histv[pl.ds(i * 16, 16)] = jnp.zeros((16,), jnp.float32)
            return carry
        lax.fori_loop(0, ROWS_PAD // 16, zbody, 0)

        @pl.when(cid == 0)
        def _():
            pltpu.sync_copy(dst0_hbm.at[sid], dstv)

        @pl.when(cid == 1)
        def _():
            pltpu.sync_copy(dst1_hbm.at[sid], dstv)

        ones = jnp.ones((16,), jnp.float32)

        def hbody(k, carry):
            idx = dstv[pl.ds(k * 16, 16)]
            plsc.addupdate_scatter(histv, [idx], ones)
            return carry
        lax.fori_loop(0, DEPT // 16, hbody, 0)

        pltpu.sync_copy(histv, stage_sh.at[sid])
        plsc.subcore_barrier()

        pltpu.sync_copy(stage_sh.at[0, pl.ds(sid * RPT, RPT)], acc)
        for j in range(1, NS):
            pltpu.sync_copy(stage_sh.at[j, pl.ds(sid * RPT, RPT)], tmp)

            def abody(i, carry):
                acc[pl.ds(i * 16, 16)] = (acc[pl.ds(i * 16, 16)]
                                          + tmp[pl.ds(i * 16, 16)])
                return carry
            lax.fori_loop(0, RPT // 16, abody, 0)

        @pl.when(cid == 0)
        def _():
            pltpu.sync_copy(acc, deg0_out.at[pl.ds(sid * RPT, RPT)])

        @pl.when(cid == 1)
        def _():
            pltpu.sync_copy(acc, deg1_out.at[pl.ds(sid * RPT, RPT)])

    return sc_degrees


_sc_degrees = _make_sc_degrees()


def _make_sc_aggregate():
    @functools.partial(
        pl.kernel,
        out_type=jax.ShapeDtypeStruct((NC, ROWS_PAD, D), jnp.float32),
        mesh=_sc_mesh,
        scratch_types=[
            pltpu.VMEM((EPT,), jnp.int32),
            pltpu.VMEM((CH,), jnp.int32),
            pltpu.VMEM((CH,), jnp.int32),
            pltpu.VMEM((CH, D), jnp.float32),
            pltpu.VMEM((CH, D), jnp.float32),
            pltpu.VMEM((CH, D), jnp.float32),
            pltpu.VMEM_SHARED((ROWS_PAD, D), jnp.float32),
            pltpu.SemaphoreType.DMA,
            pltpu.SemaphoreType.DMA,
            pltpu.SemaphoreType.DMA,
            pltpu.SemaphoreType.DMA,
        ],
    )
    def sc_aggregate(h_hbm, src_hbm, dst_hbm, agg_out,
                     srcv, dstv0, dstv1, rows0, rows1, zw_v, agg_sh,
                     semd0, semd1, semg0, semg1):
        cid = lax.axis_index("c")
        sid = lax.axis_index("s")
        wid = cid * NS + sid

        _fill_wide(zw_v, CH, 0.0)
        pltpu.sync_copy(src_hbm.at[wid], srcv)
        for j in range(RPT // CH):
            pltpu.sync_copy(zw_v, agg_sh.at[pl.ds(sid * RPT + j * CH, CH)])
        plsc.subcore_barrier()

        def start(c, dstv, semd, rows, semg):
            pltpu.async_copy(dst_hbm.at[wid * CPT + c], dstv, semd)
            pltpu.async_copy(h_hbm.at[srcv.at[pl.ds(c * CH, CH)]],
                             rows, semg)

        def finish(c, dstv, semd, rows, semg):
            pltpu.make_async_copy(dst_hbm.at[wid * CPT + c], dstv,
                                  semd).wait()
            pltpu.make_async_copy(h_hbm.at[srcv.at[pl.ds(c * CH, CH)]],
                                  rows, semg).wait()
            pltpu.sync_copy(rows, agg_sh.at[dstv], add=True)

        start(0, dstv0, semd0, rows0, semg0)

        def pair(g, carry):
            c0 = 2 * g
            start(c0 + 1, dstv1, semd1, rows1, semg1)
            finish(c0, dstv0, semd0, rows0, semg0)
            start(c0 + 2, dstv0, semd0, rows0, semg0)
            finish(c0 + 1, dstv1, semd1, rows1, semg1)
            return carry

        lax.fori_loop(0, (CPT - 1) // 2, pair, 0)
        finish(CPT - 1, dstv0, semd0, rows0, semg0)
        plsc.subcore_barrier()

        pltpu.sync_copy(agg_sh.at[pl.ds(sid * RPT, RPT)],
                        agg_out.at[cid, pl.ds(sid * RPT, RPT)])

    return sc_aggregate


_sc_aggregate = _make_sc_aggregate()


def _make_sc_pair_gather():
    @functools.partial(
        pl.kernel,
        out_type=[
            jax.ShapeDtypeStruct((P_PAD, D), jnp.float32),
            jax.ShapeDtypeStruct((P_PAD, D), jnp.float32),
        ],
        mesh=_sc_mesh,
        scratch_types=[
            pltpu.VMEM((PPT,), jnp.int32),
            pltpu.VMEM((PPT,), jnp.int32),
            pltpu.VMEM((GCH, D), jnp.float32),
            pltpu.VMEM((GCH, D), jnp.float32),
            pltpu.SemaphoreType.DMA,
            pltpu.SemaphoreType.DMA,
        ],
    )
    def sc_pair_gather(h_hbm, pu_hbm, pv_hbm, hu_out, hv_out,
                       puv, pvv, rows0, rows1, sem0, sem1):
        cid = lax.axis_index("c")
        sid = lax.axis_index("s")
        wid = cid * NS + sid

        pltpu.sync_copy(pu_hbm.at[wid], puv)
        pltpu.sync_copy(pv_hbm.at[wid], pvv)

        def start(idx_ref, c, rows, sem):
            pltpu.async_copy(h_hbm.at[idx_ref.at[pl.ds(c * GCH, GCH)]],
                             rows, sem)

        def finish(idx_ref, c, rows, sem, out):
            pltpu.make_async_copy(h_hbm.at[idx_ref.at[pl.ds(c * GCH, GCH)]],
                                  rows, sem).wait()
            pltpu.sync_copy(rows, out.at[pl.ds(wid * PPT + c * GCH, GCH)])

        start(puv, 0, rows0, sem0)

        def chunk(c, carry):
            start(pvv, c, rows1, sem1)
            finish(puv, c, rows0, sem0, hu_out)
            start(puv, c + 1, rows0, sem0)
            finish(pvv, c, rows1, sem1, hv_out)
            return carry

        lax.fori_loop(0, NGCH - 1, chunk, 0)
        start(pvv, NGCH - 1, rows1, sem1)
        finish(puv, NGCH - 1, rows0, sem0, hu_out)
        finish(pvv, NGCH - 1, rows1, sem1, hv_out)

    return sc_pair_gather


_sc_pair_gather = _make_sc_pair_gather()


def _layer_body(h_ref, aggp_ref, deg_ref, ws_ref, wn_ref, b_ref,
                g_ref, be_ref, out_ref):
    deg = jnp.maximum(deg_ref[...], 1.0)
    agg = aggp_ref[0, 0:N_NODES, :] + aggp_ref[1, 0:N_NODES, :]
    mean = agg / deg
    z = (jnp.dot(h_ref[...], ws_ref[...], preferred_element_type=jnp.float32)
         + jnp.dot(mean, wn_ref[...], preferred_element_type=jnp.float32)
         + b_ref[...])
    mu = jnp.mean(z, axis=0, keepdims=True)
    var = jnp.mean((z - mu) * (z - mu), axis=0, keepdims=True)
    bn = g_ref[...] * (z - mu) / jnp.sqrt(var + EPS) + be_ref[...]
    out_ref[...] = jnp.maximum(bn, 0.0)


def _tc_layer(h, aggp, deg, w_self, w_neigh, b, gamma, beta):
    return pl.pallas_call(
        _layer_body,
        out_shape=jax.ShapeDtypeStruct((N_NODES, D), jnp.float32),
    )(h, aggp, deg, w_self, w_neigh, b, gamma, beta)


MLP_BLK = 2048


def _mlp_body(hu_ref, hv_ref, ef_ref, wa_ref, wb_ref, wc_ref, b1_ref,
              w2_ref, b2_ref, out_ref):
    z = (jnp.dot(hu_ref[...], wa_ref[...], preferred_element_type=jnp.float32)
         + jnp.dot(hv_ref[...], wb_ref[...], preferred_element_type=jnp.float32)
         + jnp.dot(ef_ref[...], wc_ref[...], preferred_element_type=jnp.float32)
         + b1_ref[...])
    z = jnp.maximum(z, 0.0)
    out_ref[...] = (jnp.dot(z, w2_ref[...], preferred_element_type=jnp.float32)
                    + b2_ref[...])


def _tc_mlp(hu, hv, ef, wa, wb, wc, b1, w2, b2):
    n_blk = P_PAD // MLP_BLK
    return pl.pallas_call(
        _mlp_body,
        grid=(n_blk,),
        in_specs=[
            pl.BlockSpec((MLP_BLK, D), lambda i: (i, 0)),
            pl.BlockSpec((MLP_BLK, D), lambda i: (i, 0)),
            pl.BlockSpec((MLP_BLK, EDGE_IN), lambda i: (i, 0)),
            pl.BlockSpec((D, D), lambda i: (0, 0)),
            pl.BlockSpec((D, D), lambda i: (0, 0)),
            pl.BlockSpec((EDGE_IN, D), lambda i: (0, 0)),
            pl.BlockSpec((1, D), lambda i: (0, 0)),
            pl.BlockSpec((D, N_CLS), lambda i: (0, 0)),
            pl.BlockSpec((1, N_CLS), lambda i: (0, 0)),
        ],
        out_specs=pl.BlockSpec((MLP_BLK, N_CLS), lambda i: (i, 0)),
        out_shape=jax.ShapeDtypeStruct((P_PAD, N_CLS), jnp.float32),
    )(hu, hv, ef, wa, wb, wc, b1, w2, b2)


def kernel(x_nodes, block0_edge_index, block1_edge_index, pair_u, pair_v,
           e_feat, W_self0, W_neigh0, b0, gamma0, beta0,
           W_self1, W_neigh1, b1, gamma1, beta1,
           W_mlp1, b_mlp1, W_mlp2, b_mlp2):
    src0 = block0_edge_index[0].reshape(NW, EPT)
    dst0 = block0_edge_index[1].reshape(NW * CPT, CH)
    src1 = block1_edge_index[0].reshape(NW, EPT)
    dst1 = block1_edge_index[1].reshape(NW * CPT, CH)
    dst0_d = block0_edge_index[1].reshape(NS, DEPT)
    dst1_d = block1_edge_index[1].reshape(NS, DEPT)
    pad = P_PAD - N_PAIRS
    pu = jnp.concatenate([pair_u, jnp.zeros((pad,), jnp.int32)])
    pv = jnp.concatenate([pair_v, jnp.zeros((pad,), jnp.int32)])
    pu2 = pu.reshape(NW, PPT)
    pv2 = pv.reshape(NW, PPT)
    ef = jnp.concatenate([e_feat, jnp.zeros((pad, EDGE_IN), jnp.float32)])

    deg0f, deg1f = _sc_degrees(dst0_d, dst1_d)
    deg0 = deg0f[:N_NODES, None]
    deg1 = deg1f[:N_NODES, None]

    aggp0 = _sc_aggregate(x_nodes, src0, dst0)
    h1 = _tc_layer(x_nodes, aggp0, deg0, W_self0, W_neigh0,
                   b0.reshape(1, D), gamma0.reshape(1, D), beta0.reshape(1, D))

    aggp1 = _sc_aggregate(h1, src1, dst1)
    h2 = _tc_layer(h1, aggp1, deg1, W_self1, W_neigh1,
                   b1.reshape(1, D), gamma1.reshape(1, D), beta1.reshape(1, D))

    hu, hv = _sc_pair_gather(h2, pu2, pv2)
    wa = W_mlp1[0:D]
    wb = W_mlp1[D:2 * D]
    wc = W_mlp1[2 * D:]
    out = _tc_mlp(hu, hv, ef, wa, wb, wc, b_mlp1.reshape(1, -1),
                  W_mlp2, b_mlp2.reshape(1, -1))
    return out[:N_PAIRS]


# speedup vs baseline: 8.3366x; 1.5312x over previous
import functools

import jax
import jax.numpy as jnp
from jax import lax
from jax.experimental import pallas as pl
from jax.experimental.pallas import tpu as pltpu
from jax.experimental.pallas import tpu_sc as plsc

N_NODES = 10000
N_EDGES = 320000
N_PAIRS = 100000
D = 128
EDGE_IN = 16
N_CLS = 8
EPS = 1e-5

NC = 2
NS = 16
NW = NC * NS

EPT = N_EDGES // NW
CH = 80
CPT = EPT // CH
ROWS_PAD = 10240
RPT = ROWS_PAD // NS

DEPT = N_EDGES // NS
DCPT = DEPT // CH

P_PAD = 102400
PPT = P_PAD // NW
GCH = 128
NGCH = PPT // GCH

_sc_mesh = plsc.VectorSubcoreMesh(core_axis_name="c", subcore_axis_name="s")


def _fill_wide(ref, n_rows, val):
    def body(k, carry):
        ref[k // 8, pl.ds((k % 8) * 16, 16)] = jnp.full((16,), val,
                                                        jnp.float32)
        return carry
    lax.fori_loop(0, n_rows * 8, body, 0)


def _make_sc_degrees():
    @functools.partial(
        pl.kernel,
        out_type=[
            jax.ShapeDtypeStruct((ROWS_PAD,), jnp.float32),
            jax.ShapeDtypeStruct((ROWS_PAD,), jnp.float32),
        ],
        mesh=_sc_mesh,
        scratch_types=[
            pltpu.VMEM((DEPT,), jnp.int32),
            pltpu.VMEM((ROWS_PAD,), jnp.float32),
            pltpu.VMEM((RPT,), jnp.float32),
            pltpu.VMEM((RPT,), jnp.float32),
            pltpu.VMEM_SHARED((NS, ROWS_PAD), jnp.float32),
        ],
        compiler_params=pltpu.CompilerParams(needs_layout_passes=False),
    )
    def sc_degrees(dst0_hbm, dst1_hbm, deg0_out, deg1_out,
                   dstv, histv, acc, tmp, stage_sh):
        cid = lax.axis_index("c")
        sid = lax.axis_index("s")

        def zbody(i, carry):
            histv[pl.ds(i * 16, 16)] = jnp.zeros((16,), jnp.float32)
            return carry
        lax.fori_loop(0, ROWS_PAD // 16, zbody, 0)

        @pl.when(cid == 0)
        def _():
            pltpu.sync_copy(dst0_hbm.at[sid], dstv)

        @pl.when(cid == 1)
        def _():
            pltpu.sync_copy(dst1_hbm.at[sid], dstv)

        ones = jnp.ones((16,), jnp.float32)

        def hbody(k, carry):
            idx = dstv[pl.ds(k * 16, 16)]
            plsc.addupdate_scatter(histv, [idx], ones)
            return carry
        lax.fori_loop(0, DEPT // 16, hbody, 0)

        pltpu.sync_copy(histv, stage_sh.at[sid])
        plsc.subcore_barrier()

        pltpu.sync_copy(stage_sh.at[0, pl.ds(sid * RPT, RPT)], acc)
        for j in range(1, NS):
            pltpu.sync_copy(stage_sh.at[j, pl.ds(sid * RPT, RPT)], tmp)

            def abody(i, carry):
                acc[pl.ds(i * 16, 16)] = (acc[pl.ds(i * 16, 16)]
                                          + tmp[pl.ds(i * 16, 16)])
                return carry
            lax.fori_loop(0, RPT // 16, abody, 0)

        @pl.when(cid == 0)
        def _():
            pltpu.sync_copy(acc, deg0_out.at[pl.ds(sid * RPT, RPT)])

        @pl.when(cid == 1)
        def _():
            pltpu.sync_copy(acc, deg1_out.at[pl.ds(sid * RPT, RPT)])

    return sc_degrees


_sc_degrees = _make_sc_degrees()


def _make_sc_aggregate():
    @functools.partial(
        pl.kernel,
        out_type=jax.ShapeDtypeStruct((NC, ROWS_PAD, D), jnp.float32),
        mesh=_sc_mesh,
        scratch_types=[
            pltpu.VMEM((EPT,), jnp.int32),
            pltpu.VMEM((CH,), jnp.int32),
            pltpu.VMEM((CH,), jnp.int32),
            pltpu.VMEM((CH, D), jnp.float32),
            pltpu.VMEM((CH, D), jnp.float32),
            pltpu.VMEM((CH, D), jnp.float32),
            pltpu.VMEM_SHARED((ROWS_PAD, D), jnp.float32),
            pltpu.SemaphoreType.DMA,
            pltpu.SemaphoreType.DMA,
            pltpu.SemaphoreType.DMA,
            pltpu.SemaphoreType.DMA,
        ],
    )
    def sc_aggregate(h_hbm, src_hbm, dst_hbm, agg_out,
                     srcv, dstv0, dstv1, rows0, rows1, zw_v, agg_sh,
                     semd0, semd1, semg0, semg1):
        cid = lax.axis_index("c")
        sid = lax.axis_index("s")
        wid = cid * NS + sid

        _fill_wide(zw_v, CH, 0.0)
        pltpu.sync_copy(src_hbm.at[wid], srcv)
        for j in range(RPT // CH):
            pltpu.sync_copy(zw_v, agg_sh.at[pl.ds(sid * RPT + j * CH, CH)])
        plsc.subcore_barrier()

        def start(c, dstv, semd, rows, semg):
            pltpu.async_copy(dst_hbm.at[wid * CPT + c], dstv, semd)
            pltpu.async_copy(h_hbm.at[srcv.at[pl.ds(c * CH, CH)]],
                             rows, semg)

        def finish(c, dstv, semd, rows, semg):
            pltpu.make_async_copy(dst_hbm.at[wid * CPT + c], dstv,
                                  semd).wait()
            pltpu.make_async_copy(h_hbm.at[srcv.at[pl.ds(c * CH, CH)]],
                                  rows, semg).wait()
            pltpu.sync_copy(rows, agg_sh.at[dstv], add=True)

        start(0, dstv0, semd0, rows0, semg0)

        def pair(g, carry):
            c0 = 2 * g
            start(c0 + 1, dstv1, semd1, rows1, semg1)
            finish(c0, dstv0, semd0, rows0, semg0)
            start(c0 + 2, dstv0, semd0, rows0, semg0)
            finish(c0 + 1, dstv1, semd1, rows1, semg1)
            return carry

        lax.fori_loop(0, (CPT - 1) // 2, pair, 0)
        finish(CPT - 1, dstv0, semd0, rows0, semg0)
        plsc.subcore_barrier()

        pltpu.sync_copy(agg_sh.at[pl.ds(sid * RPT, RPT)],
                        agg_out.at[cid, pl.ds(sid * RPT, RPT)])

    return sc_aggregate


_sc_aggregate = _make_sc_aggregate()


def _make_sc_pair_gather():
    @functools.partial(
        pl.kernel,
        out_type=[
            jax.ShapeDtypeStruct((P_PAD, D), jnp.float32),
            jax.ShapeDtypeStruct((P_PAD, D), jnp.float32),
        ],
        mesh=_sc_mesh,
        scratch_types=[
            pltpu.VMEM((PPT,), jnp.int32),
            pltpu.VMEM((PPT,), jnp.int32),
            pltpu.VMEM((GCH, D), jnp.float32),
            pltpu.VMEM((GCH, D), jnp.float32),
            pltpu.VMEM_SHARED((ROWS_PAD, D), jnp.float32),
            pltpu.SemaphoreType.DMA,
            pltpu.SemaphoreType.DMA,
        ],
    )
    def sc_pair_gather(h_hbm, pu_hbm, pv_hbm, hu_out, hv_out,
                       puv, pvv, rows0, rows1, h_sh, sem0, sem1):
        cid = lax.axis_index("c")
        sid = lax.axis_index("s")
        wid = cid * NS + sid

        pltpu.sync_copy(pu_hbm.at[wid], puv)
        pltpu.sync_copy(pv_hbm.at[wid], pvv)

        @pl.when(sid < NS - 1)
        def _():
            pltpu.sync_copy(h_hbm.at[pl.ds(sid * RPT, RPT)],
                            h_sh.at[pl.ds(sid * RPT, RPT)])

        @pl.when(sid == NS - 1)
        def _():
            pltpu.sync_copy(h_hbm.at[pl.ds((NS - 1) * RPT, N_NODES
                                           - (NS - 1) * RPT)],
                            h_sh.at[pl.ds((NS - 1) * RPT, N_NODES
                                          - (NS - 1) * RPT)])

        plsc.subcore_barrier()

        def start(idx_ref, c, rows, sem):
            pltpu.async_copy(h_sh.at[idx_ref.at[pl.ds(c * GCH, GCH)]],
                             rows, sem)

        def finish(idx_ref, c, rows, sem, out):
            pltpu.make_async_copy(h_sh.at[idx_ref.at[pl.ds(c * GCH, GCH)]],
                                  rows, sem).wait()
            pltpu.sync_copy(rows, out.at[pl.ds(wid * PPT + c * GCH, GCH)])

        start(puv, 0, rows0, sem0)

        def chunk(c, carry):
            start(pvv, c, rows1, sem1)
            finish(puv, c, rows0, sem0, hu_out)
            start(puv, c + 1, rows0, sem0)
            finish(pvv, c, rows1, sem1, hv_out)
            return carry

        lax.fori_loop(0, NGCH - 1, chunk, 0)
        start(pvv, NGCH - 1, rows1, sem1)
        finish(puv, NGCH - 1, rows0, sem0, hu_out)
        finish(pvv, NGCH - 1, rows1, sem1, hv_out)

    return sc_pair_gather


_sc_pair_gather = _make_sc_pair_gather()


def _layer_body(h_ref, aggp_ref, deg_ref, ws_ref, wn_ref, b_ref,
                g_ref, be_ref, out_ref):
    deg = jnp.maximum(deg_ref[...], 1.0)
    agg = aggp_ref[0, 0:N_NODES, :] + aggp_ref[1, 0:N_NODES, :]
    mean = agg / deg
    z = (jnp.dot(h_ref[...], ws_ref[...], preferred_element_type=jnp.float32)
         + jnp.dot(mean, wn_ref[...], preferred_element_type=jnp.float32)
         + b_ref[...])
    mu = jnp.mean(z, axis=0, keepdims=True)
    var = jnp.mean((z - mu) * (z - mu), axis=0, keepdims=True)
    bn = g_ref[...] * (z - mu) / jnp.sqrt(var + EPS) + be_ref[...]
    out_ref[...] = jnp.maximum(bn, 0.0)


def _tc_layer(h, aggp, deg, w_self, w_neigh, b, gamma, beta):
    return pl.pallas_call(
        _layer_body,
        out_shape=jax.ShapeDtypeStruct((N_NODES, D), jnp.float32),
    )(h, aggp, deg, w_self, w_neigh, b, gamma, beta)


MLP_BLK = 2000


def _mlp_body(hu_ref, hv_ref, ef_ref, wa_ref, wb_ref, wc_ref, b1_ref,
              w2_ref, b2_ref, out_ref):
    z = (jnp.dot(hu_ref[...], wa_ref[...], preferred_element_type=jnp.float32)
         + jnp.dot(hv_ref[...], wb_ref[...], preferred_element_type=jnp.float32)
         + jnp.dot(ef_ref[...], wc_ref[...], preferred_element_type=jnp.float32)
         + b1_ref[...])
    z = jnp.maximum(z, 0.0)
    out_ref[...] = (jnp.dot(z, w2_ref[...], preferred_element_type=jnp.float32)
                    + b2_ref[...])


def _tc_mlp(hu, hv, ef, wa, wb, wc, b1, w2, b2):
    n_blk = N_PAIRS // MLP_BLK
    return pl.pallas_call(
        _mlp_body,
        grid=(n_blk,),
        in_specs=[
            pl.BlockSpec((MLP_BLK, D), lambda i: (i, 0)),
            pl.BlockSpec((MLP_BLK, D), lambda i: (i, 0)),
            pl.BlockSpec((MLP_BLK, EDGE_IN), lambda i: (i, 0)),
            pl.BlockSpec((D, D), lambda i: (0, 0)),
            pl.BlockSpec((D, D), lambda i: (0, 0)),
            pl.BlockSpec((EDGE_IN, D), lambda i: (0, 0)),
            pl.BlockSpec((1, D), lambda i: (0, 0)),
            pl.BlockSpec((D, N_CLS), lambda i: (0, 0)),
            pl.BlockSpec((1, N_CLS), lambda i: (0, 0)),
        ],
        out_specs=pl.BlockSpec((MLP_BLK, N_CLS), lambda i: (i, 0)),
        out_shape=jax.ShapeDtypeStruct((N_PAIRS, N_CLS), jnp.float32),
    )(hu, hv, ef, wa, wb, wc, b1, w2, b2)


def kernel(x_nodes, block0_edge_index, block1_edge_index, pair_u, pair_v,
           e_feat, W_self0, W_neigh0, b0, gamma0, beta0,
           W_self1, W_neigh1, b1, gamma1, beta1,
           W_mlp1, b_mlp1, W_mlp2, b_mlp2):
    src0 = block0_edge_index[0].reshape(NW, EPT)
    dst0 = block0_edge_index[1].reshape(NW * CPT, CH)
    src1 = block1_edge_index[0].reshape(NW, EPT)
    dst1 = block1_edge_index[1].reshape(NW * CPT, CH)
    dst0_d = block0_edge_index[1].reshape(NS, DEPT)
    dst1_d = block1_edge_index[1].reshape(NS, DEPT)
    pad = P_PAD - N_PAIRS
    pu = jnp.concatenate([pair_u, jnp.zeros((pad,), jnp.int32)])
    pv = jnp.concatenate([pair_v, jnp.zeros((pad,), jnp.int32)])
    pu2 = pu.reshape(NW, PPT)
    pv2 = pv.reshape(NW, PPT)

    deg0f, deg1f = _sc_degrees(dst0_d, dst1_d)
    deg0 = deg0f[:N_NODES, None]
    deg1 = deg1f[:N_NODES, None]

    aggp0 = _sc_aggregate(x_nodes, src0, dst0)
    h1 = _tc_layer(x_nodes, aggp0, deg0, W_self0, W_neigh0,
                   b0.reshape(1, D), gamma0.reshape(1, D), beta0.reshape(1, D))

    aggp1 = _sc_aggregate(h1, src1, dst1)
    h2 = _tc_layer(h1, aggp1, deg1, W_self1, W_neigh1,
                   b1.reshape(1, D), gamma1.reshape(1, D), beta1.reshape(1, D))

    hu, hv = _sc_pair_gather(h2, pu2, pv2)
    wa = W_mlp1[0:D]
    wb = W_mlp1[D:2 * D]
    wc = W_mlp1[2 * D:]
    out = _tc_mlp(hu, hv, e_feat, wa, wb, wc, b_mlp1.reshape(1, -1),
                  W_mlp2, b_mlp2.reshape(1, -1))
    return out


# speedup vs baseline: 8.3458x; 1.0011x over previous
import functools

import jax
import jax.numpy as jnp
from jax import lax
from jax.experimental import pallas as pl
from jax.experimental.pallas import tpu as pltpu
from jax.experimental.pallas import tpu_sc as plsc

N_NODES = 10000
N_EDGES = 320000
N_PAIRS = 100000
D = 128
EDGE_IN = 16
N_CLS = 8
EPS = 1e-5

NC = 2
NS = 16
NW = NC * NS

EPT = N_EDGES // NW
CH = 80
CPT = EPT // CH
NSLOT = 5
ROWS_PAD = 10240
RPT = ROWS_PAD // NS

DEPT = N_EDGES // NS
DCPT = DEPT // CH

P_PAD = 102400
PPT = P_PAD // NW
GCH = 128
NGCH = PPT // GCH

_sc_mesh = plsc.VectorSubcoreMesh(core_axis_name="c", subcore_axis_name="s")


def _fill_wide(ref, n_rows, val):
    def body(k, carry):
        ref[k // 8, pl.ds((k % 8) * 16, 16)] = jnp.full((16,), val,
                                                        jnp.float32)
        return carry
    lax.fori_loop(0, n_rows * 8, body, 0)


def _make_sc_degrees():
    @functools.partial(
        pl.kernel,
        out_type=[
            jax.ShapeDtypeStruct((ROWS_PAD,), jnp.float32),
            jax.ShapeDtypeStruct((ROWS_PAD,), jnp.float32),
        ],
        mesh=_sc_mesh,
        scratch_types=[
            pltpu.VMEM((DEPT,), jnp.int32),
            pltpu.VMEM((ROWS_PAD,), jnp.float32),
            pltpu.VMEM((RPT,), jnp.float32),
            pltpu.VMEM((RPT,), jnp.float32),
            pltpu.VMEM_SHARED((NS, ROWS_PAD), jnp.float32),
        ],
        compiler_params=pltpu.CompilerParams(needs_layout_passes=False),
    )
    def sc_degrees(dst0_hbm, dst1_hbm, deg0_out, deg1_out,
                   dstv, histv, acc, tmp, stage_sh):
        cid = lax.axis_index("c")
        sid = lax.axis_index("s")

        def zbody(i, carry):
            histv[pl.ds(i * 16, 16)] = jnp.zeros((16,), jnp.float32)
            return carry
        lax.fori_loop(0, ROWS_PAD // 16, zbody, 0)

        @pl.when(cid == 0)
        def _():
            pltpu.sync_copy(dst0_hbm.at[sid], dstv)

        @pl.when(cid == 1)
        def _():
            pltpu.sync_copy(dst1_hbm.at[sid], dstv)

        ones = jnp.ones((16,), jnp.float32)

        def hbody(k, carry):
            idx = dstv[pl.ds(k * 16, 16)]
            plsc.addupdate_scatter(histv, [idx], ones)
            return carry
        lax.fori_loop(0, DEPT // 16, hbody, 0)

        pltpu.sync_copy(histv, stage_sh.at[sid])
        plsc.subcore_barrier()

        pltpu.sync_copy(stage_sh.at[0, pl.ds(sid * RPT, RPT)], acc)
        for j in range(1, NS):
            pltpu.sync_copy(stage_sh.at[j, pl.ds(sid * RPT, RPT)], tmp)

            def abody(i, carry):
                acc[pl.ds(i * 16, 16)] = (acc[pl.ds(i * 16, 16)]
                                          + tmp[pl.ds(i * 16, 16)])
                return carry
            lax.fori_loop(0, RPT // 16, abody, 0)

        @pl.when(cid == 0)
        def _():
            pltpu.sync_copy(acc, deg0_out.at[pl.ds(sid * RPT, RPT)])

        @pl.when(cid == 1)
        def _():
            pltpu.sync_copy(acc, deg1_out.at[pl.ds(sid * RPT, RPT)])

    return sc_degrees


_sc_degrees = _make_sc_degrees()


def _make_sc_aggregate():
    @functools.partial(
        pl.kernel,
        out_type=jax.ShapeDtypeStruct((NC, ROWS_PAD, D), jnp.float32),
        mesh=_sc_mesh,
        scratch_types=[
            pltpu.VMEM((EPT,), jnp.int32),
            pltpu.VMEM((CH,), jnp.int32),
            pltpu.VMEM((CH,), jnp.int32),
            pltpu.VMEM((CH, D), jnp.float32),
            pltpu.VMEM((CH, D), jnp.float32),
            pltpu.VMEM((CH, D), jnp.float32),
            pltpu.VMEM_SHARED((ROWS_PAD, D), jnp.float32),
            pltpu.SemaphoreType.DMA,
            pltpu.SemaphoreType.DMA,
            pltpu.SemaphoreType.DMA,
            pltpu.SemaphoreType.DMA,
        ],
    )
    def sc_aggregate(h_hbm, src_hbm, dst_hbm, agg_out,
                     srcv, dstv0, dstv1, rows0, rows1, zw_v, agg_sh,
                     semd0, semd1, semg0, semg1):
        cid = lax.axis_index("c")
        sid = lax.axis_index("s")
        wid = cid * NS + sid

        _fill_wide(zw_v, CH, 0.0)
        pltpu.sync_copy(src_hbm.at[wid], srcv)
        for j in range(RPT // CH):
            pltpu.sync_copy(zw_v, agg_sh.at[pl.ds(sid * RPT + j * CH, CH)])
        plsc.subcore_barrier()

        def start(c, dstv, semd, rows, semg):
            pltpu.async_copy(dst_hbm.at[wid * CPT + c], dstv, semd)
            pltpu.async_copy(h_hbm.at[srcv.at[pl.ds(c * CH, CH)]],
                             rows, semg)

        def finish(c, dstv, semd, rows, semg):
            pltpu.make_async_copy(dst_hbm.at[wid * CPT + c], dstv,
                                  semd).wait()
            pltpu.make_async_copy(h_hbm.at[srcv.at[pl.ds(c * CH, CH)]],
                                  rows, semg).wait()
            pltpu.sync_copy(rows, agg_sh.at[dstv], add=True)

        start(0, dstv0, semd0, rows0, semg0)

        def pair(g, carry):
            c0 = 2 * g
            start(c0 + 1, dstv1, semd1, rows1, semg1)
            finish(c0, dstv0, semd0, rows0, semg0)
            start(c0 + 2, dstv0, semd0, rows0, semg0)
            finish(c0 + 1, dstv1, semd1, rows1, semg1)
            return carry

        lax.fori_loop(0, (CPT - 1) // 2, pair, 0)
        finish(CPT - 1, dstv0, semd0, rows0, semg0)
        plsc.subcore_barrier()

        pltpu.sync_copy(agg_sh.at[pl.ds(sid * RPT, RPT)],
                        agg_out.at[cid, pl.ds(sid * RPT, RPT)])

    return sc_aggregate


_sc_aggregate = _make_sc_aggregate()


def _make_sc_pair_gather():
    @functools.partial(
        pl.kernel,
        out_type=[
            jax.ShapeDtypeStruct((P_PAD, D), jnp.float32),
            jax.ShapeDtypeStruct((P_PAD, D), jnp.float32),
        ],
        mesh=_sc_mesh,
        scratch_types=[
            pltpu.VMEM((PPT,), jnp.int32),
            pltpu.VMEM((PPT,), jnp.int32),
            pltpu.VMEM((GCH, D), jnp.float32),
            pltpu.VMEM((GCH, D), jnp.float32),
            pltpu.VMEM_SHARED((ROWS_PAD, D), jnp.float32),
            pltpu.SemaphoreType.DMA,
            pltpu.SemaphoreType.DMA,
        ],
    )
    def sc_pair_gather(h_hbm, pu_hbm, pv_hbm, hu_out, hv_out,
                       puv, pvv, rows0, rows1, h_sh, sem0, sem1):
        cid = lax.axis_index("c")
        sid = lax.axis_index("s")
        wid = cid * NS + sid

        pltpu.sync_copy(pu_hbm.at[wid], puv)
        pltpu.sync_copy(pv_hbm.at[wid], pvv)

        @pl.when(sid < NS - 1)
        def _():
            pltpu.sync_copy(h_hbm.at[pl.ds(sid * RPT, RPT)],
                            h_sh.at[pl.ds(sid * RPT, RPT)])

        @pl.when(sid == NS - 1)
        def _():
            pltpu.sync_copy(h_hbm.at[pl.ds((NS - 1) * RPT, N_NODES
                                           - (NS - 1) * RPT)],
                            h_sh.at[pl.ds((NS - 1) * RPT, N_NODES
                                          - (NS - 1) * RPT)])

        plsc.subcore_barrier()

        def start(idx_ref, c, rows, sem):
            pltpu.async_copy(h_sh.at[idx_ref.at[pl.ds(c * GCH, GCH)]],
                             rows, sem)

        def finish(idx_ref, c, rows, sem, out):
            pltpu.make_async_copy(h_sh.at[idx_ref.at[pl.ds(c * GCH, GCH)]],
                                  rows, sem).wait()
            pltpu.sync_copy(rows, out.at[pl.ds(wid * PPT + c * GCH, GCH)])

        start(puv, 0, rows0, sem0)

        def chunk(c, carry):
            start(pvv, c, rows1, sem1)
            finish(puv, c, rows0, sem0, hu_out)
            start(puv, c + 1, rows0, sem0)
            finish(pvv, c, rows1, sem1, hv_out)
            return carry

        lax.fori_loop(0, NGCH - 1, chunk, 0)
        start(pvv, NGCH - 1, rows1, sem1)
        finish(puv, NGCH - 1, rows0, sem0, hu_out)
        finish(pvv, NGCH - 1, rows1, sem1, hv_out)

    return sc_pair_gather


_sc_pair_gather = _make_sc_pair_gather()


def _layer_body(h_ref, aggp_ref, deg_ref, ws_ref, wn_ref, b_ref,
                g_ref, be_ref, out_ref):
    deg = jnp.maximum(deg_ref[...], 1.0)
    agg = aggp_ref[0, 0:N_NODES, :] + aggp_ref[1, 0:N_NODES, :]
    mean = agg / deg
    z = (jnp.dot(h_ref[...], ws_ref[...], preferred_element_type=jnp.float32)
         + jnp.dot(mean, wn_ref[...], preferred_element_type=jnp.float32)
         + b_ref[...])
    mu = jnp.mean(z, axis=0, keepdims=True)
    var = jnp.mean((z - mu) * (z - mu), axis=0, keepdims=True)
    bn = g_ref[...] * (z - mu) / jnp.sqrt(var + EPS) + be_ref[...]
    out_ref[...] = jnp.maximum(bn, 0.0)


def _tc_layer(h, aggp, deg, w_self, w_neigh, b, gamma, beta):
    return pl.pallas_call(
        _layer_body,
        out_shape=jax.ShapeDtypeStruct((N_NODES, D), jnp.float32),
    )(h, aggp, deg, w_self, w_neigh, b, gamma, beta)


MLP_BLK = 2000


def _mlp_body(hu_ref, hv_ref, ef_ref, wa_ref, wb_ref, wc_ref, b1_ref,
              w2_ref, b2_ref, out_ref):
    z = (jnp.dot(hu_ref[...], wa_ref[...], preferred_element_type=jnp.float32)
         + jnp.dot(hv_ref[...], wb_ref[...], preferred_element_type=jnp.float32)
         + jnp.dot(ef_ref[...], wc_ref[...], preferred_element_type=jnp.float32)
         + b1_ref[...])
    z = jnp.maximum(z, 0.0)
    out_ref[...] = (jnp.dot(z, w2_ref[...], preferred_element_type=jnp.float32)
                    + b2_ref[...])


def _tc_mlp(hu, hv, ef, wa, wb, wc, b1, w2, b2):
    n_blk = N_PAIRS // MLP_BLK
    return pl.pallas_call(
        _mlp_body,
        grid=(n_blk,),
        in_specs=[
            pl.BlockSpec((MLP_BLK, D), lambda i: (i, 0)),
            pl.BlockSpec((MLP_BLK, D), lambda i: (i, 0)),
            pl.BlockSpec((MLP_BLK, EDGE_IN), lambda i: (i, 0)),
            pl.BlockSpec((D, D), lambda i: (0, 0)),
            pl.BlockSpec((D, D), lambda i: (0, 0)),
            pl.BlockSpec((EDGE_IN, D), lambda i: (0, 0)),
            pl.BlockSpec((1, D), lambda i: (0, 0)),
            pl.BlockSpec((D, N_CLS), lambda i: (0, 0)),
            pl.BlockSpec((1, N_CLS), lambda i: (0, 0)),
        ],
        out_specs=pl.BlockSpec((MLP_BLK, N_CLS), lambda i: (i, 0)),
        out_shape=jax.ShapeDtypeStruct((N_PAIRS, N_CLS), jnp.float32),
    )(hu, hv, ef, wa, wb, wc, b1, w2, b2)


def kernel(x_nodes, block0_edge_index, block1_edge_index, pair_u, pair_v,
           e_feat, W_self0, W_neigh0, b0, gamma0, beta0,
           W_self1, W_neigh1, b1, gamma1, beta1,
           W_mlp1, b_mlp1, W_mlp2, b_mlp2):
    src0 = block0_edge_index[0].reshape(NW, EPT)
    dst0 = block0_edge_index[1].reshape(NW * CPT, CH)
    src1 = block1_edge_index[0].reshape(NW, EPT)
    dst1 = block1_edge_index[1].reshape(NW * CPT, CH)
    dst0_d = block0_edge_index[1].reshape(NS, DEPT)
    dst1_d = block1_edge_index[1].reshape(NS, DEPT)
    pad = P_PAD - N_PAIRS
    pu = jnp.concatenate([pair_u, jnp.zeros((pad,), jnp.int32)])
    pv = jnp.concatenate([pair_v, jnp.zeros((pad,), jnp.int32)])
    pu2 = pu.reshape(NW, PPT)
    pv2 = pv.reshape(NW, PPT)

    deg0f, deg1f = _sc_degrees(dst0_d, dst1_d)
    deg0 = deg0f[:N_NODES, None]
    deg1 = deg1f[:N_NODES, None]

    aggp0 = _sc_aggregate(x_nodes, src0, dst0)
    h1 = _tc_layer(x_nodes, aggp0, deg0, W_self0, W_neigh0,
                   b0.reshape(1, D), gamma0.reshape(1, D), beta0.reshape(1, D))

    aggp1 = _sc_aggregate(h1, src1, dst1)
    h2 = _tc_layer(h1, aggp1, deg1, W_self1, W_neigh1,
                   b1.reshape(1, D), gamma1.reshape(1, D), beta1.reshape(1, D))

    hu, hv = _sc_pair_gather(h2, pu2, pv2)
    wa = W_mlp1[0:D]
    wb = W_mlp1[D:2 * D]
    wc = W_mlp1[2 * D:]
    out = _tc_mlp(hu, hv, e_feat, wa, wb, wc, b_mlp1.reshape(1, -1),
                  W_mlp2, b_mlp2.reshape(1, -1))
    return out
